# Initial kernel scaffold; baseline (speedup 1.0000x reference)
import jax
import jax.numpy as jnp
from jax.experimental import pallas as pl


def kernel(x, edge_index, W1, b1, W2, b2, W3, b3, W4, b4, W5, b5):
    raise NotImplementedError("write your pallas kernel here")


# speedup vs baseline: 13.8247x; 13.8247x over previous
import functools

import jax
import jax.numpy as jnp
from jax import lax
from jax.experimental import pallas as pl
from jax.experimental.pallas import tpu as pltpu
from jax.experimental.pallas import tpu_sc as plsc

N = 100000
E = 1600000
NP = 100352
SENT = N
CH = 128
EROWS = 12800
EPAD = EROWS * CH - E
G = 8
NC, NS = 2, 16
NW = NC * NS
SPAN = NP // NS
BR = NP // 8


def _sc_mesh():
    return plsc.VectorSubcoreMesh(core_axis_name="c", subcore_axis_name="s")


def _zero_vmem(ref, nrows):
    z = jnp.zeros((16,), jnp.float32)

    def body(i, _):
        ref[i, :] = z
        return 0

    lax.fori_loop(0, nrows, body, 0)


def _zero_vmem1(ref, n):
    z = jnp.zeros((16,), jnp.float32)

    def body(i, _):
        ref[pl.ds(i * 16, 16)] = z
        return 0

    lax.fori_loop(0, n // 16, body, 0)


def _zero_acc16(gb_v, acc_sh, sid):
    _zero_vmem(gb_v, G * CH)
    nfull, rem = SPAN // (G * CH), SPAN % (G * CH)
    for t in range(nfull):
        pltpu.sync_copy(gb_v, acc_sh.at[pl.ds(sid * SPAN + t * G * CH,
                                              G * CH)])
    if rem:
        pltpu.sync_copy(gb_v.at[pl.ds(0, rem)],
                        acc_sh.at[pl.ds(sid * SPAN + nfull * G * CH, rem)])


def _degrees(src2, dst2):
    @functools.partial(
        pl.kernel,
        out_type=jax.ShapeDtypeStruct((NC, NP), jnp.float32),
        mesh=_sc_mesh(),
        compiler_params=pltpu.CompilerParams(use_tc_tiling_on_sc=False),
        scratch_types=[
            pltpu.VMEM((G, CH), jnp.int32),
            pltpu.VMEM((CH,), jnp.float32),
            pltpu.VMEM((SPAN,), jnp.float32),
            pltpu.VMEM_SHARED((NP,), jnp.float32),
            pltpu.SemaphoreType.DMA,
        ],
    )
    def body(src_h, dst_h, out_h, idx_v, ones_v, zb_v, acc_sh, sem):
        cid = lax.axis_index("c")
        sid = lax.axis_index("s")
        one = jnp.ones((16,), jnp.float32)
        for i in range(CH // 16):
            ones_v[pl.ds(i * 16, 16)] = one
        _zero_vmem1(zb_v, SPAN)
        pltpu.sync_copy(zb_v, acc_sh.at[pl.ds(sid * SPAN, SPAN)])
        plsc.subcore_barrier()

        rows_per_sub = EROWS // NS

        def step(i, _):
            r = sid * rows_per_sub + i * G

            @pl.when(cid == 0)
            def _():
                pltpu.sync_copy(src_h.at[pl.ds(r, G)], idx_v)

            @pl.when(cid == 1)
            def _():
                pltpu.sync_copy(dst_h.at[pl.ds(r, G)], idx_v)

            for j in range(G):
                pltpu.sync_copy(ones_v, acc_sh.at[idx_v.at[j]], add=True)
            return 0

        lax.fori_loop(0, rows_per_sub // G, step, 0)
        plsc.subcore_barrier()
        pltpu.sync_copy(acc_sh.at[pl.ds(sid * SPAN, SPAN)],
                        out_h.at[cid, pl.ds(sid * SPAN, SPAN)])

    return body(src2, dst2)


def _prop1(g, src2, dst2):
    @functools.partial(
        pl.kernel,
        out_type=jax.ShapeDtypeStruct((NC, NP), jnp.float32),
        mesh=_sc_mesh(),
        compiler_params=pltpu.CompilerParams(use_tc_tiling_on_sc=False),
        scratch_types=[
            pltpu.VMEM((G, CH), jnp.int32),
            pltpu.VMEM((G, CH), jnp.int32),
            pltpu.VMEM((G, CH), jnp.float32),
            pltpu.VMEM((SPAN,), jnp.float32),
            pltpu.VMEM_SHARED((NP,), jnp.float32),
            pltpu.SemaphoreType.DMA,
        ],
    )
    def body(g_h, src_h, dst_h, out_h, si_v, di_v, gb_v, zb_v, acc_sh, sem):
        cid = lax.axis_index("c")
        sid = lax.axis_index("s")
        _zero_vmem1(zb_v, SPAN)
        pltpu.sync_copy(zb_v, acc_sh.at[pl.ds(sid * SPAN, SPAN)])
        plsc.subcore_barrier()

        wid = sid * NC + cid
        rows_per_w = EROWS // NW

        def step(i, _):
            r = wid * rows_per_w + i * G
            pltpu.sync_copy(src_h.at[pl.ds(r, G)], si_v)
            pltpu.sync_copy(dst_h.at[pl.ds(r, G)], di_v)
            cps = [pltpu.async_copy(g_h.at[si_v.at[j]], gb_v.at[j], sem)
                   for j in range(G)]
            for c in cps:
                c.wait()
            for j in range(G):
                pltpu.sync_copy(gb_v.at[j], acc_sh.at[di_v.at[j]], add=True)
            return 0

        lax.fori_loop(0, rows_per_w // G, step, 0)
        plsc.subcore_barrier()
        pltpu.sync_copy(acc_sh.at[pl.ds(sid * SPAN, SPAN)],
                        out_h.at[cid, pl.ds(sid * SPAN, SPAN)])

    return body(g, src2, dst2)


def _prop16(g, src2, dst2):
    @functools.partial(
        pl.kernel,
        out_type=jax.ShapeDtypeStruct((NC, NP, 16), jnp.float32),
        mesh=_sc_mesh(),
        compiler_params=pltpu.CompilerParams(use_tc_tiling_on_sc=False),
        scratch_types=[
            pltpu.VMEM((G, CH), jnp.int32),
            pltpu.VMEM((G, CH), jnp.int32),
            pltpu.VMEM((G * CH, 16), jnp.float32),
            pltpu.VMEM_SHARED((NP, 16), jnp.float32),
            pltpu.SemaphoreType.DMA,
        ],
    )
    def body(g_h, src_h, dst_h, out_h, si_v, di_v, gb_v, acc_sh, sem):
        cid = lax.axis_index("c")
        sid = lax.axis_index("s")
        _zero_acc16(gb_v, acc_sh, sid)
        plsc.subcore_barrier()

        wid = sid * NC + cid
        rows_per_w = EROWS // NW

        def step(i, _):
            r = wid * rows_per_w + i * G
            pltpu.sync_copy(src_h.at[pl.ds(r, G)], si_v)
            pltpu.sync_copy(dst_h.at[pl.ds(r, G)], di_v)
            cps = [pltpu.async_copy(g_h.at[si_v.at[j]],
                                    gb_v.at[pl.ds(j * CH, CH)], sem)
                   for j in range(G)]
            for c in cps:
                c.wait()
            for j in range(G):
                pltpu.sync_copy(gb_v.at[pl.ds(j * CH, CH)],
                                acc_sh.at[di_v.at[j]], add=True)
            return 0

        lax.fori_loop(0, rows_per_w // G, step, 0)
        plsc.subcore_barrier()
        pltpu.sync_copy(acc_sh.at[pl.ds(sid * SPAN, SPAN)],
                        out_h.at[cid, pl.ds(sid * SPAN, SPAN)])

    return body(g, src2, dst2)


def _prop32(g2, src2, dst2):
    @functools.partial(
        pl.kernel,
        out_type=jax.ShapeDtypeStruct((NC, NP, 16), jnp.float32),
        mesh=_sc_mesh(),
        compiler_params=pltpu.CompilerParams(use_tc_tiling_on_sc=False),
        scratch_types=[
            pltpu.VMEM((G, CH), jnp.int32),
            pltpu.VMEM((G, CH), jnp.int32),
            pltpu.VMEM((G * CH, 16), jnp.float32),
            pltpu.VMEM_SHARED((NP, 16), jnp.float32),
            pltpu.SemaphoreType.DMA,
        ],
    )
    def body(g_h, src_h, dst_h, out_h, si_v, di_v, gb_v, acc_sh, sem):
        cid = lax.axis_index("c")
        sid = lax.axis_index("s")
        _zero_acc16(gb_v, acc_sh, sid)
        plsc.subcore_barrier()

        rows_per_sub = EROWS // NS

        def step(i, _):
            r = sid * rows_per_sub + i * G
            pltpu.sync_copy(src_h.at[pl.ds(r, G)], si_v)
            pltpu.sync_copy(dst_h.at[pl.ds(r, G)], di_v)
            cps = [pltpu.async_copy(g_h.at[cid].at[si_v.at[j]],
                                    gb_v.at[pl.ds(j * CH, CH)], sem)
                   for j in range(G)]
            for c in cps:
                c.wait()
            for j in range(G):
                pltpu.sync_copy(gb_v.at[pl.ds(j * CH, CH)],
                                acc_sh.at[di_v.at[j]], add=True)
            return 0

        lax.fori_loop(0, rows_per_sub // G, step, 0)
        plsc.subcore_barrier()
        pltpu.sync_copy(acc_sh.at[pl.ds(sid * SPAN, SPAN)],
                        out_h.at[cid, pl.ds(sid * SPAN, SPAN)])

    return body(g2, src2, dst2)


BR2 = NP // 32


def _col2(d=1):
    return pl.BlockSpec((BR2, d), lambda i: (i, 0))


def _feat2(d):
    return pl.BlockSpec((NC, BR2, d), lambda i: (0, i, 0))


def _sq(nd=1):
    if nd == 1:
        return pl.BlockSpec((784, 128), lambda i: (0, 0))
    return pl.BlockSpec((NC, 784, 128), lambda i: (0, 0, 0))


def _full(shape):
    return pl.BlockSpec(shape, lambda i: tuple(0 for _ in shape))


def _tc_prep(deg_out, deg_in, xp):
    def body(do_r, di_r, x_r, ns_r, nd_r, g1_r):
        do = do_r[...]
        di = di_r[...]
        ns = lax.rsqrt(jnp.where(do > 0, do, 1.0))
        nd = lax.rsqrt(jnp.where(di > 0, di, 1.0))
        ns_r[...] = ns
        nd_r[...] = nd
        g1_r[...] = ns * x_r[...]

    out = jax.ShapeDtypeStruct((784, 128), jnp.float32)
    return pl.pallas_call(
        body, grid=(1,),
        in_specs=[_sq(), _sq(), _sq()],
        out_specs=[_sq(), _sq(), _sq()],
        out_shape=[out, out, out],
    )(deg_out, deg_in, xp)


def _tc_l1(s1, ns, nd, W1, b1):
    def body(s_r, ns_r, nd_r, w_r, b_r, o_r):
        p = nd_r[...] * (s_r[0] + s_r[1])
        h = jnp.maximum(p * w_r[...] + b_r[...], 0.0)
        o_r[...] = ns_r[...] * h

    return pl.pallas_call(
        body, grid=(32,),
        in_specs=[_feat2(1), _col2(), _col2(), _full((1, 16)),
                  _full((1, 16))],
        out_specs=_col2(16),
        out_shape=jax.ShapeDtypeStruct((NP, 16), jnp.float32),
    )(s1, ns, nd, W1, b1)


def _tc_l2(s2, ns, nd, W2, b2):
    def body(s_r, ns_r, nd_r, w_r, b_r, o_r):
        p = nd_r[...] * (s_r[0] + s_r[1])
        h = jnp.maximum(
            jnp.dot(p, w_r[...], preferred_element_type=jnp.float32)
            + b_r[...], 0.0)
        g = ns_r[...] * h
        o_r[0] = g[:, :16]
        o_r[1] = g[:, 16:]

    return pl.pallas_call(
        body, grid=(32,),
        in_specs=[_feat2(16), _col2(), _col2(), _full((16, 32)),
                  _full((1, 32))],
        out_specs=_feat2(16),
        out_shape=jax.ShapeDtypeStruct((NC, NP, 16), jnp.float32),
    )(s2, ns, nd, W2, b2)


def _tc_l3(s3, ns, nd, W3, b3, W4):
    def body(s_r, ns_r, nd_r, w3_r, b3_r, w4_r, o_r):
        p = nd_r[...] * jnp.concatenate([s_r[0], s_r[1]], axis=1)
        h = jnp.maximum(
            jnp.dot(p, w3_r[...], preferred_element_type=jnp.float32)
            + b3_r[...], 0.0)
        t = jnp.dot(h, w4_r[...], preferred_element_type=jnp.float32)
        g = ns_r[...] * t
        o_r[0] = g[:, :16]
        o_r[1] = g[:, 16:]

    return pl.pallas_call(
        body, grid=(32,),
        in_specs=[_feat2(16), _col2(), _col2(), _full((32, 64)),
                  _full((1, 64)), _full((64, 32))],
        out_specs=_feat2(16),
        out_shape=jax.ShapeDtypeStruct((NC, NP, 16), jnp.float32),
    )(s3, ns, nd, W3, b3, W4)


def _tc_l4(s4, ns, nd, b4, W5):
    def body(s_r, ns_r, nd_r, b4_r, w5_r, o_r):
        p = nd_r[...] * jnp.concatenate([s_r[0], s_r[1]], axis=1)
        h = jnp.maximum(p + b4_r[...], 0.0)
        t = jnp.dot(h, w5_r[...], preferred_element_type=jnp.float32)
        o_r[...] = ns_r[...] * t

    return pl.pallas_call(
        body, grid=(32,),
        in_specs=[_feat2(16), _col2(), _col2(), _full((1, 32)),
                  _full((32, 1))],
        out_specs=_col2(),
        out_shape=jax.ShapeDtypeStruct((NP, 1), jnp.float32),
    )(s4, ns, nd, b4, W5)


def _tc_out(s5, nd, b5):
    def body(s_r, nd_r, b_r, o_r):
        p = nd_r[...] * (s_r[0] + s_r[1]) + b_r[...]
        o_r[...] = 1.0 / (1.0 + jnp.exp(-p))

    return pl.pallas_call(
        body, grid=(1,),
        in_specs=[_sq(2), _sq(), _full((1, 1))],
        out_specs=_sq(),
        out_shape=jax.ShapeDtypeStruct((784, 128), jnp.float32),
    )(s5, nd, b5)


def kernel(x, edge_index, W1, b1, W2, b2, W3, b3, W4, b4, W5, b5):
    pad = jnp.full((2, EPAD), SENT, dtype=jnp.int32)
    ei = jnp.concatenate([edge_index.astype(jnp.int32), pad], axis=1)
    src2 = ei[0].reshape(EROWS, CH)
    dst2 = ei[1].reshape(EROWS, CH)
    xp = jnp.pad(x.reshape(-1), (0, NP - N)).reshape(784, 128)

    deg = _degrees(src2, dst2)
    ns2, nd2, g12 = _tc_prep(deg[0].reshape(784, 128),
                             deg[1].reshape(784, 128), xp)
    ns = ns2.reshape(NP, 1)
    nd = nd2.reshape(NP, 1)

    s1 = _prop1(g12.reshape(NP), src2, dst2)
    g2 = _tc_l1(s1.reshape(NC, NP, 1), ns, nd, W1, b1.reshape(1, 16))

    s2 = _prop16(g2, src2, dst2)
    g3 = _tc_l2(s2, ns, nd, W2, b2.reshape(1, 32))

    s3 = _prop32(g3, src2, dst2)
    g4 = _tc_l3(s3, ns, nd, W3, b3.reshape(1, 64), W4)

    s4 = _prop32(g4, src2, dst2)
    g5 = _tc_l4(s4, ns, nd, b4.reshape(1, 32), W5)

    s5 = _prop1(g5.reshape(NP), src2, dst2)
    out = _tc_out(s5.reshape(NC, 784, 128), nd2, b5.reshape(1, 1))

    return out.reshape(NP)[:N].reshape(1, N)
